# baseline (speedup 1.0000x reference)
import functools
import math

import jax
import jax.numpy as jnp
from jax import lax
from jax.experimental import pallas as pl
from jax.experimental.pallas import tpu as pltpu
from jax.experimental.pallas import tpu_sc as plsc

_LOG_SLOPE = math.log(0.001)
_N = 256
_ROWS = 16384
_COLS = 512
_LANES = 16
_NWORKERS = 16
_RPW = _ROWS // _NWORKERS
_BLK_ROWS = 4096


def _logd_sc_body(s_hbm, d_hbm, sv, dv, sem_d):
    wid = lax.axis_index("s")
    base = wid * _RPW

    pltpu.sync_copy(s_hbm, sv)
    acc = jnp.zeros((_LANES,), jnp.float32)
    for k in range(_N // _LANES):
        s = sv[pl.ds(k * _LANES, _LANES)]
        acc = acc + s / (1.0 + jnp.abs(s * (1.0 / _LOG_SLOPE)))
    total = acc[0]
    for i in range(1, _LANES):
        total = total + acc[i]

    dvec = jnp.full((_LANES,), total, dtype=jnp.float32)
    for i in range(_RPW // _LANES):
        dv[pl.ds(i * _LANES, _LANES)] = dvec
    pltpu.make_async_copy(dv, d_hbm.at[pl.ds(base, _RPW)], sem_d).start()
    pltpu.make_async_copy(dv, d_hbm.at[pl.ds(base, _RPW)], sem_d).wait()


_logd_sc = functools.partial(
    pl.kernel,
    out_type=jax.ShapeDtypeStruct((_ROWS,), jnp.float32),
    mesh=plsc.VectorSubcoreMesh(core_axis_name="c", subcore_axis_name="s",
                                num_cores=1),
    scratch_types=[
        pltpu.VMEM((_N,), jnp.float32),
        pltpu.VMEM((_RPW,), jnp.float32),
        pltpu.SemaphoreType.DMA,
    ],
)(_logd_sc_body)


def _affine_tc_body(u_ref, s_ref, t_ref, x_ref):
    s = s_ref[0, :]
    logs = s / (1.0 + jnp.abs(s * (1.0 / _LOG_SLOPE)))
    scale = jnp.exp(logs)
    x_ref[:, :_N] = u_ref[:, :_N] * scale[None, :] + t_ref[0, :][None, :]
    x_ref[:, _N:] = u_ref[:, _N:]


@jax.jit
def kernel(u, un_s, t):
    x = pl.pallas_call(
        _affine_tc_body,
        grid=(_ROWS // _BLK_ROWS,),
        in_specs=[
            pl.BlockSpec((_BLK_ROWS, _COLS), lambda i: (i, 0)),
            pl.BlockSpec((1, _N), lambda i: (0, 0)),
            pl.BlockSpec((1, _N), lambda i: (0, 0)),
        ],
        out_specs=pl.BlockSpec((_BLK_ROWS, _COLS), lambda i: (i, 0)),
        out_shape=jax.ShapeDtypeStruct((_ROWS, _COLS), jnp.float32),
    )(u, un_s.reshape(1, _N), t.reshape(1, _N))
    logd = _logd_sc(un_s)
    return (x, logd)


# speedup vs baseline: 1.0085x; 1.0085x over previous
import functools
import math

import jax
import jax.numpy as jnp
from jax import lax
from jax.experimental import pallas as pl
from jax.experimental.pallas import tpu as pltpu
from jax.experimental.pallas import tpu_sc as plsc

_LOG_SLOPE = math.log(0.001)
_N = 256
_ROWS = 16384
_COLS = 512
_LANES = 16
_NWORKERS = 4
_RPW = _ROWS // _NWORKERS
_BLK_ROWS = 4096


def _logd_sc_body(s_hbm, d_hbm, sv, dv, sem_d):
    wid = lax.axis_index("s")
    base = wid * _RPW

    pltpu.sync_copy(s_hbm, sv)
    acc = jnp.zeros((_LANES,), jnp.float32)
    for k in range(_N // _LANES):
        s = sv[pl.ds(k * _LANES, _LANES)]
        acc = acc + s / (1.0 + jnp.abs(s * (1.0 / _LOG_SLOPE)))
    total = acc[0]
    for i in range(1, _LANES):
        total = total + acc[i]

    dvec = jnp.full((_LANES,), total, dtype=jnp.float32)
    for i in range(_RPW // _LANES):
        dv[pl.ds(i * _LANES, _LANES)] = dvec
    pltpu.make_async_copy(dv, d_hbm.at[pl.ds(base, _RPW)], sem_d).start()
    pltpu.make_async_copy(dv, d_hbm.at[pl.ds(base, _RPW)], sem_d).wait()


_logd_sc = functools.partial(
    pl.kernel,
    out_type=jax.ShapeDtypeStruct((_ROWS,), jnp.float32),
    mesh=plsc.VectorSubcoreMesh(core_axis_name="c", subcore_axis_name="s",
                                num_cores=1, num_subcores=_NWORKERS),
    scratch_types=[
        pltpu.VMEM((_N,), jnp.float32),
        pltpu.VMEM((_RPW,), jnp.float32),
        pltpu.SemaphoreType.DMA,
    ],
)(_logd_sc_body)


def _affine_tc_body(u_ref, s_ref, t_ref, x_ref):
    s = s_ref[0, :]
    logs = s / (1.0 + jnp.abs(s * (1.0 / _LOG_SLOPE)))
    scale = jnp.exp(logs)
    x_ref[:, :_N] = u_ref[:, :_N] * scale[None, :] + t_ref[0, :][None, :]
    x_ref[:, _N:] = u_ref[:, _N:]


@jax.jit
def kernel(u, un_s, t):
    x = pl.pallas_call(
        _affine_tc_body,
        grid=(_ROWS // _BLK_ROWS,),
        in_specs=[
            pl.BlockSpec((_BLK_ROWS, _COLS), lambda i: (i, 0)),
            pl.BlockSpec((1, _N), lambda i: (0, 0)),
            pl.BlockSpec((1, _N), lambda i: (0, 0)),
        ],
        out_specs=pl.BlockSpec((_BLK_ROWS, _COLS), lambda i: (i, 0)),
        out_shape=jax.ShapeDtypeStruct((_ROWS, _COLS), jnp.float32),
    )(u, un_s.reshape(1, _N), t.reshape(1, _N))
    logd = _logd_sc(un_s)
    return (x, logd)
